# baseline (speedup 1.0000x reference)
import jax
import jax.numpy as jnp
from jax.experimental import pallas as pl
from jax.experimental.pallas import tpu as pltpu

TOKENS = 8192
D_MODEL = 4096
D_HID = 1024
N_EXPERTS = 64

BLK_M = 512
N_BLK = TOKENS // BLK_M


def _gate_kernel(x_ref, w1_ref, b1_ref, w2_ref, b2_ref, out_ref, h2):
    i = pl.program_id(0)

    h_prev = h2[(i + 1) % 2]
    logits = jnp.dot(h_prev, w2_ref[...],
                     preferred_element_type=jnp.float32) + b2_ref[...]
    m = jnp.max(logits, axis=-1, keepdims=True)
    e = jnp.exp(logits - m)
    out_ref[...] = e / jnp.sum(e, axis=-1, keepdims=True)

    h = jnp.dot(x_ref[...], w1_ref[...], preferred_element_type=jnp.float32)
    h2[i % 2] = jnp.maximum(h + b1_ref[...], 0.0)


@jax.jit
def kernel(x, W1, b1, W2, b2):
    b1_2d = b1.reshape(1, D_HID)
    b2_2d = b2.reshape(1, N_EXPERTS)
    grid = (N_BLK + 1,)
    return pl.pallas_call(
        _gate_kernel,
        grid=grid,
        in_specs=[
            pl.BlockSpec((BLK_M, D_MODEL),
                         lambda i: (jnp.minimum(i, N_BLK - 1), 0)),
            pl.BlockSpec((D_MODEL, D_HID), lambda i: (0, 0)),
            pl.BlockSpec((1, D_HID), lambda i: (0, 0)),
            pl.BlockSpec((D_HID, N_EXPERTS), lambda i: (0, 0)),
            pl.BlockSpec((1, N_EXPERTS), lambda i: (0, 0)),
        ],
        out_specs=pl.BlockSpec((BLK_M, N_EXPERTS),
                               lambda i: (jnp.maximum(i - 1, 0), 0)),
        out_shape=jax.ShapeDtypeStruct((TOKENS, N_EXPERTS), jnp.float32),
        scratch_shapes=[pltpu.VMEM((2, BLK_M, D_HID), jnp.float32)],
    )(x, W1, b1_2d, W2, b2_2d)
